# Initial kernel scaffold; baseline (speedup 1.0000x reference)
import jax
import jax.numpy as jnp
from jax.experimental import pallas as pl


def kernel(x, edge_index, edge_attn, W1_0, b1_0, W2_0, b2_0, W1_1, b1_1, W2_1, b2_1):
    raise NotImplementedError("write your pallas kernel here")


# speedup vs baseline: 4.1515x; 4.1515x over previous
import functools

import jax
import jax.numpy as jnp
from jax import lax
from jax.experimental import pallas as pl
from jax.experimental.pallas import tpu as pltpu
from jax.experimental.pallas import tpu_sc as plsc

N = 10000
E = 320000
D = 128

NC = 2
NS = 16
DH = D // NC
EPT = E // NS
CH = 80
NCHUNK = EPT // CH
N2 = 10240
NPT = N2 // NS
ZR = 80

_mesh = plsc.VectorSubcoreMesh(
    core_axis_name="c", subcore_axis_name="s", num_cores=NC, num_subcores=NS)

_gdn = lax.GatherDimensionNumbers(
    offset_dims=(), collapsed_slice_dims=(0,), start_index_map=(0,))


def _splat(vec, l):
  idx = jnp.full((16,), l, dtype=jnp.int32)
  return lax.gather(vec, idx[:, None], _gdn, (1,),
                    mode=lax.GatherScatterMode.PROMISE_IN_BOUNDS)


@functools.partial(
    pl.kernel,
    out_type=jax.ShapeDtypeStruct((NC, N2, DH), jnp.float32),
    mesh=_mesh,
    scratch_types=[
        pltpu.VMEM((NCHUNK, CH), jnp.int32),
        pltpu.VMEM((NCHUNK, CH), jnp.int32),
        pltpu.VMEM((EPT,), jnp.float32),
        pltpu.VMEM((CH, DH), jnp.float32),
        pltpu.VMEM((ZR, DH), jnp.float32),
        pltpu.VMEM_SHARED((N2, DH), jnp.float32),
        pltpu.SemaphoreType.DMA,
    ],
    compiler_params=pltpu.CompilerParams(use_tc_tiling_on_sc=False),
)
def _sc_segsum(hs_hbm, src_hbm, dst_hbm, attn_hbm, zeros_hbm, part_hbm,
               src_v, dst_v, attn_v, rows_v, zbuf_v, acc_sh, sem):
  c = lax.axis_index("c")
  s = lax.axis_index("s")

  pltpu.sync_copy(zeros_hbm, zbuf_v)
  for r in range(NPT // ZR):
    pltpu.sync_copy(zbuf_v, acc_sh.at[pl.ds(s * NPT + r * ZR, ZR)])
  plsc.subcore_barrier()

  pltpu.sync_copy(src_hbm.at[s], src_v)
  pltpu.sync_copy(dst_hbm.at[s], dst_v)
  pltpu.sync_copy(attn_hbm.at[s], attn_v)

  h_half = hs_hbm.at[c]

  def chunk(i, carry):
    pltpu.async_copy(h_half.at[src_v.at[i]], rows_v, sem).wait()
    for g in range(CH // 16):
      a16 = attn_v[pl.ds(i * CH + g * 16, 16)]
      for l in range(16):
        asp = _splat(a16, l)
        e = g * 16 + l
        for j in range(DH // 16):
          rows_v[e, pl.ds(j * 16, 16)] = rows_v[e, pl.ds(j * 16, 16)] * asp
    pltpu.async_copy(rows_v, acc_sh.at[dst_v.at[i]], sem, add=True).wait()
    return carry

  lax.fori_loop(0, NCHUNK, chunk, 0)
  plsc.subcore_barrier()

  pltpu.sync_copy(acc_sh.at[pl.ds(s * NPT, NPT)],
                  part_hbm.at[c, pl.ds(s * NPT, NPT)])


_ROWS = 400
_NBLK = N // _ROWS


def _bi_body(h_ref, p0_ref, p1_ref, w1_ref, b1_ref, w2_ref, b2_ref, o_ref):
  h = h_ref[...]
  hn = jnp.concatenate([p0_ref[...], p1_ref[...]], axis=1)
  cn = (((1,), (1,)), ((), ()))
  t1 = lax.dot_general(h + hn, w1_ref[...], cn,
                       preferred_element_type=jnp.float32) + b1_ref[...]
  t2 = lax.dot_general(h * hn, w2_ref[...], cn,
                       preferred_element_type=jnp.float32) + b2_ref[...]
  o_ref[...] = jnp.where(t1 > 0, t1, 0.01 * t1) + jnp.where(t2 > 0, t2, 0.01 * t2)


def _bi(h, p0, p1, w1, b1, w2, b2):
  rspec = lambda w: pl.BlockSpec((_ROWS, w), lambda i: (i, 0))
  wspec = pl.BlockSpec((D, D), lambda i: (0, 0))
  bspec = pl.BlockSpec((1, D), lambda i: (0, 0))
  return pl.pallas_call(
      _bi_body,
      grid=(_NBLK,),
      in_specs=[rspec(D), rspec(DH), rspec(DH), wspec, bspec, wspec, bspec],
      out_specs=rspec(D),
      out_shape=jax.ShapeDtypeStruct((N, D), jnp.float32),
  )(h, p0, p1, w1, b1.reshape(1, D), w2, b2.reshape(1, D))


def kernel(x, edge_index, edge_attn,
           W1_0, b1_0, W2_0, b2_0, W1_1, b1_1, W2_1, b2_1):
  src = edge_index[0].reshape(NS, NCHUNK, CH)
  dst = edge_index[1].reshape(NS, NCHUNK, CH)
  attn = edge_attn.reshape(NS, EPT)
  zeros = jnp.zeros((ZR, DH), jnp.float32)

  def split(h):
    return h.reshape(N, NC, DH).transpose(1, 0, 2)

  part = _sc_segsum(split(x), src, dst, attn, zeros)
  h1 = _bi(x, part[0, :N], part[1, :N], W1_0, b1_0, W2_0, b2_0)
  part = _sc_segsum(split(h1), src, dst, attn, zeros)
  h2 = _bi(h1, part[0, :N], part[1, :N], W1_1, b1_1, W2_1, b2_1)
  return jnp.concatenate([x, h1, h2], axis=1)


# speedup vs baseline: 5.7030x; 1.3737x over previous
import functools

import jax
import jax.numpy as jnp
from jax import lax
from jax.experimental import pallas as pl
from jax.experimental.pallas import tpu as pltpu
from jax.experimental.pallas import tpu_sc as plsc

N = 10000
E = 320000
D = 128

NC = 2
NS = 16
DH = D // NC
EPT = E // NS
CH = 80
NCHUNK = EPT // CH
N2 = 10240
NPT = N2 // NS
ZR = 80

_mesh = plsc.VectorSubcoreMesh(
    core_axis_name="c", subcore_axis_name="s", num_cores=NC, num_subcores=NS)

_gdn = lax.GatherDimensionNumbers(
    offset_dims=(), collapsed_slice_dims=(0,), start_index_map=(0,))


def _splat(vec, l):
  idx = jnp.full((16,), l, dtype=jnp.int32)
  return lax.gather(vec, idx[:, None], _gdn, (1,),
                    mode=lax.GatherScatterMode.PROMISE_IN_BOUNDS)


@functools.partial(
    pl.kernel,
    out_type=jax.ShapeDtypeStruct((NC, N2, DH), jnp.float32),
    mesh=_mesh,
    scratch_types=[
        pltpu.VMEM((NCHUNK, CH), jnp.int32),
        pltpu.VMEM((NCHUNK, CH), jnp.int32),
        pltpu.VMEM((EPT,), jnp.float32),
        pltpu.VMEM((CH, DH), jnp.float32),
        pltpu.VMEM((CH, DH), jnp.float32),
        pltpu.VMEM((ZR, DH), jnp.float32),
        pltpu.VMEM_SHARED((N2, DH), jnp.float32),
        pltpu.SemaphoreType.DMA,
        pltpu.SemaphoreType.DMA,
    ],
    compiler_params=pltpu.CompilerParams(use_tc_tiling_on_sc=False),
)
def _sc_segsum(hs_hbm, src_hbm, dst_hbm, attn_hbm, zeros_hbm, part_hbm,
               src_v, dst_v, attn_v, rows0_v, rows1_v, zbuf_v, acc_sh,
               gsem, ssem):
  c = lax.axis_index("c")
  s = lax.axis_index("s")

  pltpu.sync_copy(zeros_hbm, zbuf_v)
  for r in range(NPT // ZR):
    pltpu.sync_copy(zbuf_v, acc_sh.at[pl.ds(s * NPT + r * ZR, ZR)])
  plsc.subcore_barrier()

  pltpu.sync_copy(src_hbm.at[s], src_v)
  pltpu.sync_copy(dst_hbm.at[s], dst_v)
  pltpu.sync_copy(attn_hbm.at[s], attn_v)

  h_half = hs_hbm.at[c]

  def gather(i, buf):
    pltpu.async_copy(h_half.at[src_v.at[i]], buf, gsem)

  def scatter(i, buf):
    pltpu.async_copy(buf, acc_sh.at[dst_v.at[i]], ssem, add=True)

  def wait_gather(buf):
    pltpu.make_async_copy(h_half.at[src_v.at[0]], buf, gsem).wait()

  def wait_scatter(buf):
    pltpu.make_async_copy(buf, acc_sh.at[dst_v.at[0]], ssem).wait()

  def scale(i, buf):
    for g in range(CH // 16):
      a16 = attn_v[pl.ds(i * CH + g * 16, 16)]
      for l in range(16):
        asp = _splat(a16, l)
        e = g * 16 + l
        for j in range(DH // 16):
          buf[e, pl.ds(j * 16, 16)] = buf[e, pl.ds(j * 16, 16)] * asp

  gather(0, rows0_v)

  def body2(t, carry):
    a = 2 * t
    b = a + 1
    wait_gather(rows0_v)

    @pl.when(t > 0)
    def _():
      wait_scatter(rows1_v)

    gather(b, rows1_v)
    scale(a, rows0_v)
    scatter(a, rows0_v)

    wait_gather(rows1_v)
    wait_scatter(rows0_v)

    @pl.when(b + 1 < NCHUNK)
    def _():
      gather(b + 1, rows0_v)

    scale(b, rows1_v)
    scatter(b, rows1_v)
    return carry

  lax.fori_loop(0, NCHUNK // 2, body2, 0)
  wait_scatter(rows1_v)
  plsc.subcore_barrier()

  pltpu.sync_copy(acc_sh.at[pl.ds(s * NPT, NPT)],
                  part_hbm.at[c, pl.ds(s * NPT, NPT)])


_ROWS = 400
_NBLK = N // _ROWS


def _bi_body(h_ref, p0_ref, p1_ref, w1_ref, b1_ref, w2_ref, b2_ref, o_ref):
  h = h_ref[...]
  hn = jnp.concatenate([p0_ref[...], p1_ref[...]], axis=1)
  cn = (((1,), (1,)), ((), ()))
  t1 = lax.dot_general(h + hn, w1_ref[...], cn,
                       preferred_element_type=jnp.float32) + b1_ref[...]
  t2 = lax.dot_general(h * hn, w2_ref[...], cn,
                       preferred_element_type=jnp.float32) + b2_ref[...]
  o_ref[...] = jnp.where(t1 > 0, t1, 0.01 * t1) + jnp.where(t2 > 0, t2, 0.01 * t2)


def _bi(h, p0, p1, w1, b1, w2, b2):
  rspec = lambda w: pl.BlockSpec((_ROWS, w), lambda i: (i, 0))
  wspec = pl.BlockSpec((D, D), lambda i: (0, 0))
  bspec = pl.BlockSpec((1, D), lambda i: (0, 0))
  return pl.pallas_call(
      _bi_body,
      grid=(_NBLK,),
      in_specs=[rspec(D), rspec(DH), rspec(DH), wspec, bspec, wspec, bspec],
      out_specs=rspec(D),
      out_shape=jax.ShapeDtypeStruct((N, D), jnp.float32),
  )(h, p0, p1, w1, b1.reshape(1, D), w2, b2.reshape(1, D))


def kernel(x, edge_index, edge_attn,
           W1_0, b1_0, W2_0, b2_0, W1_1, b1_1, W2_1, b2_1):
  src = edge_index[0].reshape(NS, NCHUNK, CH)
  dst = edge_index[1].reshape(NS, NCHUNK, CH)
  attn = edge_attn.reshape(NS, EPT)
  zeros = jnp.zeros((ZR, DH), jnp.float32)

  def split(h):
    return h.reshape(N, NC, DH).transpose(1, 0, 2)

  part = _sc_segsum(split(x), src, dst, attn, zeros)
  h1 = _bi(x, part[0, :N], part[1, :N], W1_0, b1_0, W2_0, b2_0)
  part = _sc_segsum(split(h1), src, dst, attn, zeros)
  h2 = _bi(h1, part[0, :N], part[1, :N], W1_1, b1_1, W2_1, b2_1)
  return jnp.concatenate([x, h1, h2], axis=1)
